# baseline (speedup 1.0000x reference)
import functools

import jax
import jax.numpy as jnp
from jax import lax
from jax.experimental import pallas as pl
from jax.experimental.pallas import tpu as pltpu
from jax.experimental.pallas import tpu_sc as plsc

B = 1024
CTX = 20
E = 64
V = 100000

NC, NS = 2, 16
NW = NC * NS
B_PER_W = B // NW
R_PER_W = B_PER_W * CTX
CHUNK = 128
NCHUNK = R_PER_W // CHUNK

LANE = 16
EV = E // LANE


def _pool_sc(x3, emb_table):
    mesh = plsc.VectorSubcoreMesh(core_axis_name="c", subcore_axis_name="s")

    @functools.partial(
        pl.kernel,
        out_type=jax.ShapeDtypeStruct((B, E), jnp.float32),
        mesh=mesh,
        scratch_types=[
            pltpu.VMEM((NCHUNK, CHUNK), jnp.int32),
            pltpu.VMEM((R_PER_W, E), jnp.float32),
            pltpu.VMEM((B_PER_W, E), jnp.float32),
            pltpu.SemaphoreType.DMA,
        ],
        compiler_params=pltpu.CompilerParams(use_tc_tiling_on_sc=False),
    )
    def k(x_hbm, tbl_hbm, out_hbm, idx_v, rows_v, pooled_v, sem):
        wid = lax.axis_index("s") * NC + lax.axis_index("c")
        pltpu.sync_copy(x_hbm.at[wid], idx_v)
        copies = [
            pltpu.async_copy(
                tbl_hbm.at[idx_v.at[j]],
                rows_v.at[pl.ds(j * CHUNK, CHUNK)],
                sem,
            )
            for j in range(NCHUNK)
        ]
        for c in copies:
            c.wait()

        inv = jnp.float32(1.0 / CTX)

        def body_e(e, carry):
            def body_c(c, acc):
                r = e * CTX + c
                return tuple(
                    acc[v] + rows_v[r, pl.ds(v * LANE, LANE)] for v in range(EV)
                )

            acc = lax.fori_loop(
                0, CTX, body_c,
                tuple(jnp.zeros((LANE,), jnp.float32) for _ in range(EV)),
            )
            for v in range(EV):
                pooled_v[e, pl.ds(v * LANE, LANE)] = acc[v] * inv
            return carry

        lax.fori_loop(0, B_PER_W, body_e, 0)
        pltpu.sync_copy(pooled_v, out_hbm.at[pl.ds(wid * B_PER_W, B_PER_W)])

    return k(x3, emb_table)


NB = 2048


def _project_tc(embeds, W):
    def mm(e_ref, w_ref, o_ref):
        o_ref[...] = lax.dot_general(
            e_ref[...], w_ref[...],
            (((1,), (1,)), ((), ())),
            preferred_element_type=jnp.float32,
        )

    return pl.pallas_call(
        mm,
        grid=(pl.cdiv(V, NB),),
        in_specs=[
            pl.BlockSpec((B, E), lambda j: (0, 0)),
            pl.BlockSpec((NB, E), lambda j: (j, 0)),
        ],
        out_specs=pl.BlockSpec((B, NB), lambda j: (0, j)),
        out_shape=jax.ShapeDtypeStruct((B, V), jnp.float32),
    )(embeds, W)


def kernel(x, emb_table, W):
    x3 = x.reshape(NW, NCHUNK, CHUNK)
    embeds = _pool_sc(x3, emb_table)
    return _project_tc(embeds, W)


# speedup vs baseline: 1.0297x; 1.0297x over previous
import functools

import jax
import jax.numpy as jnp
from jax import lax
from jax.experimental import pallas as pl
from jax.experimental.pallas import tpu as pltpu
from jax.experimental.pallas import tpu_sc as plsc

B = 1024
CTX = 20
E = 64
V = 100000

NC, NS = 2, 16
NW = NC * NS
B_PER_W = B // NW
R_PER_W = B_PER_W * CTX
CHUNK = 128
NCHUNK = R_PER_W // CHUNK

LANE = 16
EV = E // LANE


def _pool_sc(x3, emb_table):
    mesh = plsc.VectorSubcoreMesh(core_axis_name="c", subcore_axis_name="s")

    @functools.partial(
        pl.kernel,
        out_type=jax.ShapeDtypeStruct((B, E), jnp.float32),
        mesh=mesh,
        scratch_types=[
            pltpu.VMEM((NCHUNK, CHUNK), jnp.int32),
            pltpu.VMEM((R_PER_W, E), jnp.float32),
            pltpu.VMEM((B_PER_W, E), jnp.float32),
            pltpu.SemaphoreType.DMA,
        ],
        compiler_params=pltpu.CompilerParams(use_tc_tiling_on_sc=False),
    )
    def k(x_hbm, tbl_hbm, out_hbm, idx_v, rows_v, pooled_v, sem):
        wid = lax.axis_index("s") * NC + lax.axis_index("c")
        pltpu.sync_copy(x_hbm.at[wid], idx_v)
        copies = [
            pltpu.async_copy(
                tbl_hbm.at[idx_v.at[j]],
                rows_v.at[pl.ds(j * CHUNK, CHUNK)],
                sem,
            )
            for j in range(NCHUNK)
        ]
        for c in copies:
            c.wait()

        inv = jnp.float32(1.0 / CTX)

        def body_e(e, carry):
            def body_c(c, acc):
                r = e * CTX + c
                return tuple(
                    acc[v] + rows_v[r, pl.ds(v * LANE, LANE)] for v in range(EV)
                )

            acc = lax.fori_loop(
                0, CTX, body_c,
                tuple(jnp.zeros((LANE,), jnp.float32) for _ in range(EV)),
            )
            for v in range(EV):
                pooled_v[e, pl.ds(v * LANE, LANE)] = acc[v] * inv
            return carry

        lax.fori_loop(0, B_PER_W, body_e, 0)
        pltpu.sync_copy(pooled_v, out_hbm.at[pl.ds(wid * B_PER_W, B_PER_W)])

    return k(x3, emb_table)


NB = 2048


def _project_tc(embeds, W):
    def mm(e_ref, w_ref, o_ref):
        o_ref[...] = lax.dot_general(
            e_ref[...], w_ref[...],
            (((1,), (1,)), ((), ())),
            preferred_element_type=jnp.float32,
        )

    return pl.pallas_call(
        mm,
        grid=(pl.cdiv(V, NB),),
        in_specs=[
            pl.BlockSpec((B, E), lambda j: (0, 0)),
            pl.BlockSpec((NB, E), lambda j: (j, 0)),
        ],
        out_specs=pl.BlockSpec((B, NB), lambda j: (0, j)),
        out_shape=jax.ShapeDtypeStruct((B, V), jnp.float32),
    )(embeds, W)


def kernel(x, emb_table, W):
    embeds = jnp.take(emb_table, x, axis=0).mean(axis=1)
    return _project_tc(embeds, W)


# speedup vs baseline: 1.0344x; 1.0046x over previous
import functools

import jax
import jax.numpy as jnp
from jax import lax
from jax.experimental import pallas as pl
from jax.experimental.pallas import tpu as pltpu
from jax.experimental.pallas import tpu_sc as plsc

B = 1024
CTX = 20
E = 64
V = 100000

NC, NS = 2, 16
NW = NC * NS
B_PER_W = B // NW
R_PER_W = B_PER_W * CTX
CHUNK = 128
NCHUNK = R_PER_W // CHUNK

LANE = 16
EV = E // LANE


def _pool_sc(x3, emb_table):
    mesh = plsc.VectorSubcoreMesh(core_axis_name="c", subcore_axis_name="s")

    @functools.partial(
        pl.kernel,
        out_type=jax.ShapeDtypeStruct((B, E), jnp.float32),
        mesh=mesh,
        scratch_types=[
            pltpu.VMEM((NCHUNK, CHUNK), jnp.int32),
            pltpu.VMEM((R_PER_W, E), jnp.float32),
            pltpu.VMEM((B_PER_W, E), jnp.float32),
            pltpu.SemaphoreType.DMA,
        ],
        compiler_params=pltpu.CompilerParams(use_tc_tiling_on_sc=False),
    )
    def k(x_hbm, tbl_hbm, out_hbm, idx_v, rows_v, pooled_v, sem):
        wid = lax.axis_index("s") * NC + lax.axis_index("c")
        pltpu.sync_copy(x_hbm.at[wid], idx_v)
        copies = [
            pltpu.async_copy(
                tbl_hbm.at[idx_v.at[j]],
                rows_v.at[pl.ds(j * CHUNK, CHUNK)],
                sem,
            )
            for j in range(NCHUNK)
        ]
        for c in copies:
            c.wait()

        inv = jnp.float32(1.0 / CTX)

        def body_e(e, carry):
            def body_c(c, acc):
                r = e * CTX + c
                return tuple(
                    acc[v] + rows_v[r, pl.ds(v * LANE, LANE)] for v in range(EV)
                )

            acc = lax.fori_loop(
                0, CTX, body_c,
                tuple(jnp.zeros((LANE,), jnp.float32) for _ in range(EV)),
            )
            for v in range(EV):
                pooled_v[e, pl.ds(v * LANE, LANE)] = acc[v] * inv
            return carry

        lax.fori_loop(0, B_PER_W, body_e, 0)
        pltpu.sync_copy(pooled_v, out_hbm.at[pl.ds(wid * B_PER_W, B_PER_W)])

    return k(x3, emb_table)


NB = 4096


def _project_tc(embeds, W):
    def mm(e_ref, w_ref, o_ref):
        o_ref[...] = lax.dot_general(
            e_ref[...], w_ref[...],
            (((1,), (1,)), ((), ())),
            preferred_element_type=jnp.float32,
        )

    return pl.pallas_call(
        mm,
        grid=(pl.cdiv(V, NB),),
        in_specs=[
            pl.BlockSpec((B, E), lambda j: (0, 0)),
            pl.BlockSpec((NB, E), lambda j: (j, 0)),
        ],
        out_specs=pl.BlockSpec((B, NB), lambda j: (0, j)),
        out_shape=jax.ShapeDtypeStruct((B, V), jnp.float32),
    )(embeds, W)


def kernel(x, emb_table, W):
    embeds = jnp.take(emb_table, x, axis=0).mean(axis=1)
    return _project_tc(embeds, W)


# speedup vs baseline: 2.3020x; 2.2254x over previous
import functools

import jax
import jax.numpy as jnp
from jax import lax
from jax.experimental import pallas as pl
from jax.experimental.pallas import tpu as pltpu
from jax.experimental.pallas import tpu_sc as plsc

B = 1024
CTX = 20
E = 64
V = 100000

NC, NS = 2, 16
NW = NC * NS
B_PER_W = B // NW
R_PER_W = B_PER_W * CTX
CHUNK = 128
NCHUNK = R_PER_W // CHUNK

LANE = 16
EV = E // LANE


def _pool_sc(x3, emb_table):
    mesh = plsc.VectorSubcoreMesh(core_axis_name="c", subcore_axis_name="s")

    @functools.partial(
        pl.kernel,
        out_type=jax.ShapeDtypeStruct((B, E), jnp.float32),
        mesh=mesh,
        scratch_types=[
            pltpu.VMEM((NCHUNK, CHUNK), jnp.int32),
            pltpu.VMEM((R_PER_W, E), jnp.float32),
            pltpu.VMEM((B_PER_W, E), jnp.float32),
            pltpu.SemaphoreType.DMA,
        ],
        compiler_params=pltpu.CompilerParams(use_tc_tiling_on_sc=False),
    )
    def k(x_hbm, tbl_hbm, out_hbm, idx_v, rows_v, pooled_v, sem):
        wid = lax.axis_index("s") * NC + lax.axis_index("c")
        pltpu.sync_copy(x_hbm.at[wid], idx_v)
        copies = [
            pltpu.async_copy(
                tbl_hbm.at[idx_v.at[j]],
                rows_v.at[pl.ds(j * CHUNK, CHUNK)],
                sem,
            )
            for j in range(NCHUNK)
        ]
        for c in copies:
            c.wait()

        inv = jnp.float32(1.0 / CTX)

        def body_e(e, carry):
            def body_c(c, acc):
                r = e * CTX + c
                return tuple(
                    acc[v] + rows_v[r, pl.ds(v * LANE, LANE)] for v in range(EV)
                )

            acc = lax.fori_loop(
                0, CTX, body_c,
                tuple(jnp.zeros((LANE,), jnp.float32) for _ in range(EV)),
            )
            for v in range(EV):
                pooled_v[e, pl.ds(v * LANE, LANE)] = acc[v] * inv
            return carry

        lax.fori_loop(0, B_PER_W, body_e, 0)
        pltpu.sync_copy(pooled_v, out_hbm.at[pl.ds(wid * B_PER_W, B_PER_W)])

    return k(x3, emb_table)


NB = 4096


def _project_tc(embeds, W):
    def mm(e_ref, w_ref, o_ref):
        o_ref[...] = lax.dot_general(
            e_ref[...], w_ref[...],
            (((1,), (1,)), ((), ())),
            preferred_element_type=jnp.float32,
        )

    return pl.pallas_call(
        mm,
        grid=(pl.cdiv(V, NB),),
        in_specs=[
            pl.BlockSpec((B, E), lambda j: (0, 0)),
            pl.BlockSpec((NB, E), lambda j: (j, 0)),
        ],
        out_specs=pl.BlockSpec((B, NB), lambda j: (0, j)),
        out_shape=jax.ShapeDtypeStruct((B, V), jnp.float32),
    )(embeds, W)


def _project_tc4(embeds, W):
    KSPLIT = 4
    nb = 1024
    nj = 25

    def mm(e_ref, w0, w1, w2, w3, o0, o1, o2, o3):
        for w_ref, o_ref in ((w0, o0), (w1, o1), (w2, o2), (w3, o3)):
            o_ref[...] = lax.dot_general(
                e_ref[...], w_ref[...],
                (((1,), (1,)), ((), ())),
                preferred_element_type=jnp.float32,
            )

    def wspec(k):
        return pl.BlockSpec((nb, E), lambda j, k=k: (k * nj + j, 0))

    return pl.pallas_call(
        mm,
        grid=(nj,),
        in_specs=[pl.BlockSpec((B, E), lambda j: (0, 0))] + [wspec(k) for k in range(KSPLIT)],
        out_specs=[pl.BlockSpec((B, nb), lambda j: (0, j)) for _ in range(KSPLIT)],
        out_shape=[jax.ShapeDtypeStruct((B, nj * nb), jnp.float32) for _ in range(KSPLIT)],
    )(embeds, W, W, W, W)


def kernel(x, emb_table, W):
    embeds = jnp.take(emb_table, x, axis=0).mean(axis=1)
    Wp = jnp.pad(W, ((0, 4 * 25 * 1024 - V), (0, 0)))
    return _project_tc4(embeds, Wp)
